# baseline (speedup 1.0000x reference)
import functools

import jax
import jax.numpy as jnp
from jax import lax
from jax.experimental import pallas as pl
from jax.experimental.pallas import tpu as pltpu
from jax.experimental.pallas import tpu_sc as plsc

NUM_EMBEDDINGS = 1000000
EMBEDDING_DIM = 64
BATCH = 16384
N_FIELDS = 26

B_FLAT = BATCH * N_FIELDS
NC, NS = 2, 16
NW = NC * NS
B_PER_W = B_FLAT // NW
CHUNK = 512
NBUF = 2
NCHUNK = B_PER_W // CHUNK


def _gather_kernel(idx_hbm, table_hbm, out_hbm, idx_v, rows_v, sems):
    wid = lax.axis_index("s") * NC + lax.axis_index("c")
    base = wid * B_PER_W

    pltpu.sync_copy(idx_hbm.at[pl.ds(base, B_PER_W)], idx_v)

    for b in range(NBUF):
        pltpu.async_copy(
            table_hbm.at[idx_v.at[pl.ds(b * CHUNK, CHUNK)]],
            rows_v.at[b],
            sems.at[b],
        )

    def step(i, _):
        for b in range(NBUF):
            chunk = i + b
            pltpu.make_async_copy(
                table_hbm.at[idx_v.at[pl.ds(chunk * CHUNK, CHUNK)]],
                rows_v.at[b],
                sems.at[b],
            ).wait()
            pltpu.sync_copy(
                rows_v.at[b], out_hbm.at[pl.ds(base + chunk * CHUNK, CHUNK)]
            )
            @pl.when(chunk + NBUF < NCHUNK)
            def _():
                pltpu.async_copy(
                    table_hbm.at[idx_v.at[pl.ds((chunk + NBUF) * CHUNK, CHUNK)]],
                    rows_v.at[b],
                    sems.at[b],
                )
        return ()

    lax.fori_loop(0, NCHUNK // NBUF, lambda i, c: step(i * NBUF, c), ())


@jax.jit
def _embedding_gather(idx_flat, weight):
    mesh = plsc.VectorSubcoreMesh(core_axis_name="c", subcore_axis_name="s")
    return pl.kernel(
        _gather_kernel,
        out_type=jax.ShapeDtypeStruct((B_FLAT, EMBEDDING_DIM), jnp.float32),
        mesh=mesh,
        scratch_types=[
            pltpu.VMEM((B_PER_W,), jnp.int32),
            pltpu.VMEM((NBUF, CHUNK, EMBEDDING_DIM), jnp.float32),
            pltpu.SemaphoreType.DMA((NBUF,)),
        ],
        compiler_params=pltpu.CompilerParams(use_tc_tiling_on_sc=False),
    )(idx_flat, weight)


def kernel(input_, num_global_tokens, weight):
    del num_global_tokens
    idx_flat = input_.reshape(-1).astype(jnp.int32)
    out = _embedding_gather(idx_flat, weight)
    return out.reshape(input_.shape + (EMBEDDING_DIM,))
